# baseline (speedup 1.0000x reference)
import functools

import jax
import jax.numpy as jnp
import numpy as np
from jax.experimental import pallas as pl
from jax.experimental.pallas import tpu as pltpu

_OBJECT_PROBS = (0.1, 0.1, 0.1, 0.1, 0.15, 0.15, 0.05, 0.05)
_COLS_PER_OBJ = 32
_MASK_VALUE = 0.0


def _threefry2x32_pair(keypair, x0, x1):
    def rotl(v, d):
        return ((v << np.uint32(d)) | (v >> np.uint32(32 - d))).astype(np.uint32)

    x = [np.asarray(x0, np.uint32).copy(), np.asarray(x1, np.uint32).copy()]
    rotations = ((13, 15, 26, 6), (17, 29, 16, 24))
    k0, k1 = np.uint32(keypair[0]), np.uint32(keypair[1])
    ks = [k0, k1, k0 ^ k1 ^ np.uint32(0x1BD11BDA)]
    x[0] = (x[0] + ks[0]).astype(np.uint32)
    x[1] = (x[1] + ks[1]).astype(np.uint32)
    for i in range(5):
        for r in rotations[i % 2]:
            x[0] = (x[0] + x[1]).astype(np.uint32)
            x[1] = rotl(x[1], r)
            x[1] = x[1] ^ x[0]
        x[0] = (x[0] + ks[(i + 1) % 3]).astype(np.uint32)
        x[1] = (x[1] + ks[(i + 2) % 3] + np.uint32(i + 1)).astype(np.uint32)
    return x


def _fold_in(keypair, i):
    o = _threefry2x32_pair(keypair, np.array([0], np.uint32), np.array([i], np.uint32))
    return np.uint32(o[0][0]), np.uint32(o[1][0])


def _np_uniform(keypair, n):
    idx = np.arange(n, dtype=np.uint64)
    o = _threefry2x32_pair(keypair, (idx >> np.uint64(32)).astype(np.uint32),
                           idx.astype(np.uint32))
    bits = o[0] ^ o[1]
    return ((bits >> np.uint32(9)) | np.uint32(0x3F800000)).view(np.float32) - np.float32(1.0)


@functools.lru_cache(maxsize=None)
def _mask_bits(batch):
    root = (np.uint32(0), np.uint32(42))
    bits = np.zeros((batch,), np.int32)
    for i, p in enumerate(_OBJECT_PROBS):
        m = _np_uniform(_fold_in(root, i), batch) < np.float32(p)
        bits |= m.astype(np.int32) << i
    return bits.reshape(batch, 1)


def _mask_kernel(bits_ref, x_ref, o_ref):
    x = x_ref[...]
    bits = bits_ref[...]
    obj = jax.lax.broadcasted_iota(jnp.int32, x.shape, 1) // _COLS_PER_OBJ
    masked = (jnp.right_shift(bits, obj) & 1) != 0
    o_ref[...] = jnp.where(masked, jnp.float32(_MASK_VALUE), x)


def _tc_mask(x):
    b, f = x.shape
    bits = jnp.asarray(_mask_bits(b))
    rows = 8192
    return pl.pallas_call(
        _mask_kernel,
        grid=(b // rows,),
        in_specs=[
            pl.BlockSpec((rows, 1), lambda i: (i, 0)),
            pl.BlockSpec((rows, f), lambda i: (i, 0)),
        ],
        out_specs=pl.BlockSpec((rows, f), lambda i: (i, 0)),
        out_shape=jax.ShapeDtypeStruct((b, f), x.dtype),
        compiler_params=pltpu.CompilerParams(
            dimension_semantics=("parallel",),
        ),
    )(bits, x)


_NW = 32
_CHUNK = 32768
_LANES = 16


@functools.lru_cache(maxsize=None)
def _sc_consts(batch, n_feat):
    n_obj = len(_OBJECT_PROBS)
    seg_w = n_feat // n_obj
    bits = _mask_bits(batch).ravel()
    rows, objs = np.nonzero((bits[:, None] >> np.arange(n_obj)) & 1)
    addr = (rows.astype(np.int64) * n_feat + objs * seg_w)
    per_w = (batch * n_feat) // _NW
    nch = per_w // _CHUNK
    lists = [[None] * nch for _ in range(_NW)]
    for w in range(_NW):
        for t in range(nch):
            lo = w * per_w + t * _CHUNK
            sel = addr[(addr >= lo) & (addr < lo + _CHUNK)] - lo
            assert len(sel) > 0
            lists[w][t] = sel.astype(np.int32)
    g_max = max(-(-len(l) // _LANES) for row in lists for l in row)
    idx = np.empty((_NW, nch * g_max * _LANES), np.int32)
    for w in range(_NW):
        for t in range(nch):
            l = lists[w][t]
            pad = np.full(g_max * _LANES, l[0], np.int32)
            pad[: len(l)] = l
            idx[w, t * g_max * _LANES: (t + 1) * g_max * _LANES] = pad
    return idx, nch, g_max


def _sc_mask(x):
    b, f = x.shape
    n_obj = len(_OBJECT_PROBS)
    seg_w = f // n_obj
    total = b * f
    per_w = total // _NW
    idx_np, nch, g_max = _sc_consts(b, f)

    from jax.experimental.pallas import tpu_sc as plsc

    mesh = plsc.VectorSubcoreMesh(core_axis_name="c", subcore_axis_name="s")
    n_cores = mesh.num_cores

    @functools.partial(
        pl.kernel,
        out_type=jax.ShapeDtypeStruct((total,), jnp.float32),
        mesh=mesh,
        scratch_types=[
            pltpu.VMEM((_CHUNK,), jnp.float32),
            pltpu.VMEM((idx_np.shape[1],), jnp.int32),
        ],
        compiler_params=pltpu.CompilerParams(needs_layout_passes=False),
    )
    def sc_kernel(x_hbm, idx_hbm, out_hbm, buf, idxv):
        wid = jax.lax.axis_index("s") * n_cores + jax.lax.axis_index("c")
        base = wid * per_w
        pltpu.sync_copy(idx_hbm.at[wid], idxv)
        zeros = jnp.zeros((_LANES,), jnp.float32)
        for t in range(nch):
            off = base + t * _CHUNK
            pltpu.sync_copy(x_hbm.at[pl.ds(off, _CHUNK)], buf)
            for g in range(g_max):
                a = idxv[pl.ds((t * g_max + g) * _LANES, _LANES)]
                for c in range(seg_w):
                    plsc.store_scatter(buf, [a + c], zeros)
            pltpu.sync_copy(buf, out_hbm.at[pl.ds(off, _CHUNK)])

    out = sc_kernel(x.reshape(total), jnp.asarray(idx_np))
    return out.reshape(b, f)


def kernel(x):
    return _sc_mask(x)


# speedup vs baseline: 1.0703x; 1.0703x over previous
import functools

import jax
import jax.numpy as jnp
import numpy as np
from jax.experimental import pallas as pl
from jax.experimental.pallas import tpu as pltpu

_OBJECT_PROBS = (0.1, 0.1, 0.1, 0.1, 0.15, 0.15, 0.05, 0.05)
_COLS_PER_OBJ = 32
_MASK_VALUE = 0.0


def _threefry2x32_pair(keypair, x0, x1):
    def rotl(v, d):
        return ((v << np.uint32(d)) | (v >> np.uint32(32 - d))).astype(np.uint32)

    x = [np.asarray(x0, np.uint32).copy(), np.asarray(x1, np.uint32).copy()]
    rotations = ((13, 15, 26, 6), (17, 29, 16, 24))
    k0, k1 = np.uint32(keypair[0]), np.uint32(keypair[1])
    ks = [k0, k1, k0 ^ k1 ^ np.uint32(0x1BD11BDA)]
    x[0] = (x[0] + ks[0]).astype(np.uint32)
    x[1] = (x[1] + ks[1]).astype(np.uint32)
    for i in range(5):
        for r in rotations[i % 2]:
            x[0] = (x[0] + x[1]).astype(np.uint32)
            x[1] = rotl(x[1], r)
            x[1] = x[1] ^ x[0]
        x[0] = (x[0] + ks[(i + 1) % 3]).astype(np.uint32)
        x[1] = (x[1] + ks[(i + 2) % 3] + np.uint32(i + 1)).astype(np.uint32)
    return x


def _fold_in(keypair, i):
    o = _threefry2x32_pair(keypair, np.array([0], np.uint32), np.array([i], np.uint32))
    return np.uint32(o[0][0]), np.uint32(o[1][0])


def _np_uniform(keypair, n):
    idx = np.arange(n, dtype=np.uint64)
    o = _threefry2x32_pair(keypair, (idx >> np.uint64(32)).astype(np.uint32),
                           idx.astype(np.uint32))
    bits = o[0] ^ o[1]
    return ((bits >> np.uint32(9)) | np.uint32(0x3F800000)).view(np.float32) - np.float32(1.0)


@functools.lru_cache(maxsize=None)
def _mask_bits(batch):
    root = (np.uint32(0), np.uint32(42))
    bits = np.zeros((batch,), np.int32)
    for i, p in enumerate(_OBJECT_PROBS):
        m = _np_uniform(_fold_in(root, i), batch) < np.float32(p)
        bits |= m.astype(np.int32) << i
    return bits.reshape(batch, 1)


def _mask_kernel(bits_ref, x_ref, o_ref):
    x = x_ref[...]
    bits = bits_ref[...]
    obj = jax.lax.broadcasted_iota(jnp.int32, x.shape, 1) // _COLS_PER_OBJ
    masked = (jnp.right_shift(bits, obj) & 1) != 0
    o_ref[...] = jnp.where(masked, jnp.float32(_MASK_VALUE), x)


def _tc_mask(x):
    b, f = x.shape
    bits = jnp.asarray(_mask_bits(b))
    rows = 8192
    return pl.pallas_call(
        _mask_kernel,
        grid=(b // rows,),
        in_specs=[
            pl.BlockSpec((rows, 1), lambda i: (i, 0)),
            pl.BlockSpec((rows, f), lambda i: (i, 0)),
        ],
        out_specs=pl.BlockSpec((rows, f), lambda i: (i, 0)),
        out_shape=jax.ShapeDtypeStruct((b, f), x.dtype),
        compiler_params=pltpu.CompilerParams(
            dimension_semantics=("parallel",),
        ),
    )(bits, x)


_NW = 32
_CHUNK = 32768
_LANES = 16


@functools.lru_cache(maxsize=None)
def _sc_consts(batch, n_feat):
    n_obj = len(_OBJECT_PROBS)
    seg_w = n_feat // n_obj
    bits = _mask_bits(batch).ravel()
    rows, objs = np.nonzero((bits[:, None] >> np.arange(n_obj)) & 1)
    addr = (rows.astype(np.int64) * n_feat + objs * seg_w)
    per_w = (batch * n_feat) // _NW
    nch = per_w // _CHUNK
    lists = [[None] * nch for _ in range(_NW)]
    for w in range(_NW):
        for t in range(nch):
            lo = w * per_w + t * _CHUNK
            sel = addr[(addr >= lo) & (addr < lo + _CHUNK)] - lo
            assert len(sel) > 0
            lists[w][t] = sel.astype(np.int32)
    g_max = max(-(-len(l) // _LANES) for row in lists for l in row)
    idx = np.empty((_NW, nch * g_max * _LANES), np.int32)
    for w in range(_NW):
        for t in range(nch):
            l = lists[w][t]
            pad = np.full(g_max * _LANES, l[0], np.int32)
            pad[: len(l)] = l
            idx[w, t * g_max * _LANES: (t + 1) * g_max * _LANES] = pad
    return idx, nch, g_max


def _sc_mask(x):
    b, f = x.shape
    n_obj = len(_OBJECT_PROBS)
    seg_w = f // n_obj
    total = b * f
    per_w = total // _NW
    idx_np, nch, g_max = _sc_consts(b, f)

    from jax.experimental.pallas import tpu_sc as plsc

    mesh = plsc.VectorSubcoreMesh(core_axis_name="c", subcore_axis_name="s")
    n_cores = mesh.num_cores

    @functools.partial(
        pl.kernel,
        out_type=jax.ShapeDtypeStruct((total,), jnp.float32),
        mesh=mesh,
        scratch_types=[
            pltpu.VMEM((_CHUNK,), jnp.float32),
            pltpu.VMEM((_CHUNK,), jnp.float32),
            pltpu.VMEM((idx_np.shape[1],), jnp.int32),
            pltpu.SemaphoreType.DMA,
            pltpu.SemaphoreType.DMA,
            pltpu.SemaphoreType.DMA,
            pltpu.SemaphoreType.DMA,
        ],
        compiler_params=pltpu.CompilerParams(needs_layout_passes=False),
    )
    def sc_kernel(x_hbm, idx_hbm, out_hbm, buf0, buf1, idxv, l0, l1, s0, s1):
        wid = jax.lax.axis_index("s") * n_cores + jax.lax.axis_index("c")
        base = wid * per_w
        bufs, lsems, ssems = (buf0, buf1), (l0, l1), (s0, s1)
        pltpu.sync_copy(idx_hbm.at[wid], idxv)
        zeros = jnp.zeros((_LANES,), jnp.float32)

        def load(t):
            return pltpu.make_async_copy(
                x_hbm.at[pl.ds(base + t * _CHUNK, _CHUNK)], bufs[t % 2], lsems[t % 2])

        def store(t):
            return pltpu.make_async_copy(
                bufs[t % 2], out_hbm.at[pl.ds(base + t * _CHUNK, _CHUNK)], ssems[t % 2])

        load(0).start()
        for t in range(nch):
            if t + 1 < nch:
                if t >= 1:
                    store(t - 1).wait()
                load(t + 1).start()
            load(t).wait()
            for g in range(g_max):
                a = idxv[pl.ds((t * g_max + g) * _LANES, _LANES)]
                for c in range(seg_w):
                    plsc.store_scatter(bufs[t % 2], [a + c], zeros)
            store(t).start()
        if nch >= 2:
            store(nch - 2).wait()
        store(nch - 1).wait()

    out = sc_kernel(x.reshape(total), jnp.asarray(idx_np))
    return out.reshape(b, f)


def kernel(x):
    return _sc_mask(x)


# speedup vs baseline: 1.8778x; 1.7545x over previous
import functools

import jax
import jax.numpy as jnp
import numpy as np
from jax.experimental import pallas as pl
from jax.experimental.pallas import tpu as pltpu

_OBJECT_PROBS = (0.1, 0.1, 0.1, 0.1, 0.15, 0.15, 0.05, 0.05)
_COLS_PER_OBJ = 32
_MASK_VALUE = 0.0


def _threefry2x32_pair(keypair, x0, x1):
    def rotl(v, d):
        return ((v << np.uint32(d)) | (v >> np.uint32(32 - d))).astype(np.uint32)

    x = [np.asarray(x0, np.uint32).copy(), np.asarray(x1, np.uint32).copy()]
    rotations = ((13, 15, 26, 6), (17, 29, 16, 24))
    k0, k1 = np.uint32(keypair[0]), np.uint32(keypair[1])
    ks = [k0, k1, k0 ^ k1 ^ np.uint32(0x1BD11BDA)]
    x[0] = (x[0] + ks[0]).astype(np.uint32)
    x[1] = (x[1] + ks[1]).astype(np.uint32)
    for i in range(5):
        for r in rotations[i % 2]:
            x[0] = (x[0] + x[1]).astype(np.uint32)
            x[1] = rotl(x[1], r)
            x[1] = x[1] ^ x[0]
        x[0] = (x[0] + ks[(i + 1) % 3]).astype(np.uint32)
        x[1] = (x[1] + ks[(i + 2) % 3] + np.uint32(i + 1)).astype(np.uint32)
    return x


def _fold_in(keypair, i):
    o = _threefry2x32_pair(keypair, np.array([0], np.uint32), np.array([i], np.uint32))
    return np.uint32(o[0][0]), np.uint32(o[1][0])


def _np_uniform(keypair, n):
    idx = np.arange(n, dtype=np.uint64)
    o = _threefry2x32_pair(keypair, (idx >> np.uint64(32)).astype(np.uint32),
                           idx.astype(np.uint32))
    bits = o[0] ^ o[1]
    return ((bits >> np.uint32(9)) | np.uint32(0x3F800000)).view(np.float32) - np.float32(1.0)


@functools.lru_cache(maxsize=None)
def _mask_bits(batch):
    root = (np.uint32(0), np.uint32(42))
    bits = np.zeros((batch,), np.int32)
    for i, p in enumerate(_OBJECT_PROBS):
        m = _np_uniform(_fold_in(root, i), batch) < np.float32(p)
        bits |= m.astype(np.int32) << i
    return bits.reshape(batch, 1)


def _mask_kernel(bits_ref, x_ref, o_ref):
    x = x_ref[...]
    bits = bits_ref[...]
    obj = jax.lax.broadcasted_iota(jnp.int32, x.shape, 1) // _COLS_PER_OBJ
    masked = (jnp.right_shift(bits, obj) & 1) != 0
    o_ref[...] = jnp.where(masked, jnp.float32(_MASK_VALUE), x)


def _tc_mask(x):
    b, f = x.shape
    bits = jnp.asarray(_mask_bits(b))
    rows = 8192
    return pl.pallas_call(
        _mask_kernel,
        grid=(b // rows,),
        in_specs=[
            pl.BlockSpec((rows, 1), lambda i: (i, 0)),
            pl.BlockSpec((rows, f), lambda i: (i, 0)),
        ],
        out_specs=pl.BlockSpec((rows, f), lambda i: (i, 0)),
        out_shape=jax.ShapeDtypeStruct((b, f), x.dtype),
        compiler_params=pltpu.CompilerParams(
            dimension_semantics=("parallel",),
        ),
    )(bits, x)


_NW = 32
_CHUNK_ROWS = 128
_LANES = 16


@functools.lru_cache(maxsize=None)
def _sc_consts(batch, n_feat):
    n_obj = len(_OBJECT_PROBS)
    seg_w = n_feat // n_obj
    bits = _mask_bits(batch).ravel()
    rows, objs = np.nonzero((bits[:, None] >> np.arange(n_obj)) & 1)
    rows = rows.astype(np.int32)
    cols = (objs * seg_w).astype(np.int32)
    rows_per_w = batch // _NW
    nch = rows_per_w // _CHUNK_ROWS
    lists = [[None] * nch for _ in range(_NW)]
    for w in range(_NW):
        for t in range(nch):
            lo = w * rows_per_w + t * _CHUNK_ROWS
            sel = (rows >= lo) & (rows < lo + _CHUNK_ROWS)
            assert sel.any()
            lists[w][t] = (rows[sel] - lo, cols[sel])
    g_max = max(-(-len(l[0]) // _LANES) for row in lists for l in row)
    ridx = np.empty((_NW, nch * g_max * _LANES), np.int32)
    cidx = np.empty((_NW, nch * g_max * _LANES), np.int32)
    for w in range(_NW):
        for t in range(nch):
            r, c = lists[w][t]
            sl = slice(t * g_max * _LANES, (t + 1) * g_max * _LANES)
            rp = np.full(g_max * _LANES, r[0], np.int32)
            cp = np.full(g_max * _LANES, c[0], np.int32)
            rp[: len(r)] = r
            cp[: len(c)] = c
            ridx[w, sl] = rp
            cidx[w, sl] = cp
    return ridx, cidx, nch, g_max


def _sc_mask(x):
    b, f = x.shape
    n_obj = len(_OBJECT_PROBS)
    seg_w = f // n_obj
    rows_per_w = b // _NW
    ridx_np, cidx_np, nch, g_max = _sc_consts(b, f)

    from jax.experimental.pallas import tpu_sc as plsc

    mesh = plsc.VectorSubcoreMesh(core_axis_name="c", subcore_axis_name="s")
    n_cores = mesh.num_cores

    @functools.partial(
        pl.kernel,
        out_type=jax.ShapeDtypeStruct((b, f), jnp.float32),
        mesh=mesh,
        scratch_types=[
            pltpu.VMEM((_CHUNK_ROWS, f), jnp.float32),
            pltpu.VMEM((_CHUNK_ROWS, f), jnp.float32),
            pltpu.VMEM((ridx_np.shape[1],), jnp.int32),
            pltpu.VMEM((cidx_np.shape[1],), jnp.int32),
            pltpu.SemaphoreType.DMA,
            pltpu.SemaphoreType.DMA,
            pltpu.SemaphoreType.DMA,
            pltpu.SemaphoreType.DMA,
        ],
        compiler_params=pltpu.CompilerParams(needs_layout_passes=False),
    )
    def sc_kernel(x_hbm, ridx_hbm, cidx_hbm, out_hbm,
                  buf0, buf1, rv, cv, l0, l1, s0, s1):
        wid = jax.lax.axis_index("s") * n_cores + jax.lax.axis_index("c")
        base = wid * rows_per_w
        bufs, lsems, ssems = (buf0, buf1), (l0, l1), (s0, s1)
        pltpu.sync_copy(ridx_hbm.at[wid], rv)
        pltpu.sync_copy(cidx_hbm.at[wid], cv)
        zeros = jnp.zeros((_LANES,), jnp.float32)

        def load(t):
            return pltpu.make_async_copy(
                x_hbm.at[pl.ds(base + t * _CHUNK_ROWS, _CHUNK_ROWS)],
                bufs[t % 2], lsems[t % 2])

        def store(t):
            return pltpu.make_async_copy(
                bufs[t % 2],
                out_hbm.at[pl.ds(base + t * _CHUNK_ROWS, _CHUNK_ROWS)],
                ssems[t % 2])

        load(0).start()
        for t in range(nch):
            if t + 1 < nch:
                if t >= 1:
                    store(t - 1).wait()
                load(t + 1).start()
            load(t).wait()
            for g in range(g_max):
                o = (t * g_max + g) * _LANES
                a = rv[pl.ds(o, _LANES)]
                cbase = cv[pl.ds(o, _LANES)]
                for c in range(seg_w):
                    plsc.store_scatter(bufs[t % 2], [a, cbase + c], zeros)
            store(t).start()
        if nch >= 2:
            store(nch - 2).wait()
        store(nch - 1).wait()

    return sc_kernel(x, jnp.asarray(ridx_np), jnp.asarray(cidx_np))


def kernel(x):
    return _sc_mask(x)
